# baseline (speedup 1.0000x reference)
import functools

import jax
import jax.numpy as jnp
from jax import lax
from jax.experimental import pallas as pl
from jax.experimental.pallas import tpu as pltpu
from jax.experimental.pallas import tpu_sc as plsc

NUM_CORES = 2
NUM_SUBCORES = 16
NUM_W = NUM_CORES * NUM_SUBCORES
CHUNK = 128
VEC = 16


def _sc_gather_pairs(idx3, table2):
    n_w, n_chunks, _ = idx3.shape
    _, d2 = table2.shape
    b_per_w = n_chunks * CHUNK
    batch = n_w * b_per_w

    mesh = plsc.VectorSubcoreMesh(core_axis_name="c", subcore_axis_name="s")

    @functools.partial(
        pl.kernel,
        mesh=mesh,
        out_type=jax.ShapeDtypeStruct((batch, d2), jnp.float32),
        scratch_types=[
            pltpu.VMEM((n_chunks, CHUNK), jnp.int32),
            pltpu.VMEM((n_chunks, CHUNK), jnp.int32),
            pltpu.VMEM((CHUNK, d2), jnp.float32),
            pltpu.SemaphoreType.DMA,
        ],
    )
    def emb(idx_hbm, table_hbm, out_hbm, idx_v, idxhi_v, rows_v, sem):
        wid = lax.axis_index("s") * NUM_CORES + lax.axis_index("c")
        base = wid * b_per_w
        pltpu.sync_copy(idx_hbm.at[wid], idx_v)
        for c in range(n_chunks):
            for k in range(CHUNK // VEC):
                sl = pl.ds(k * VEC, VEC)
                idxhi_v.at[c][sl] = lax.shift_right_logical(
                    idx_v.at[c][sl], 1
                )
        for c in range(n_chunks):
            pltpu.async_copy(
                table_hbm.at[idxhi_v.at[c]], rows_v, sem
            ).wait()
            pltpu.sync_copy(
                rows_v, out_hbm.at[pl.ds(base + c * CHUNK, CHUNK)]
            )

    return emb(idx3, table2)


def _tc_select_kernel(sel_ref, pairs_ref, out_ref):
    odd = (sel_ref[...] & 1).astype(jnp.float32)
    pairs = pairs_ref[...]
    lo = pairs[:, :64]
    hi = pairs[:, 64:]
    out_ref[...] = lo + (hi - lo) * odd


def _tc_select(type_id2, pairs):
    batch = pairs.shape[0]
    block_b = 2048
    grid = (batch // block_b,)
    return pl.pallas_call(
        _tc_select_kernel,
        grid=grid,
        in_specs=[
            pl.BlockSpec((block_b, 1), lambda i: (i, 0)),
            pl.BlockSpec((block_b, 128), lambda i: (i, 0)),
        ],
        out_specs=pl.BlockSpec((block_b, 64), lambda i: (i, 0)),
        out_shape=jax.ShapeDtypeStruct((batch, 64), jnp.float32),
    )(type_id2, pairs)


def kernel(type_id, table):
    B = type_id.shape[0]
    V, D = table.shape
    b_per_w = B // NUM_W
    n_chunks = b_per_w // CHUNK

    idx3 = type_id.reshape(NUM_W, n_chunks, CHUNK)
    table2 = table.reshape(V // 2, 2 * D)

    pairs = _sc_gather_pairs(idx3, table2)
    return _tc_select(type_id.reshape(B, 1), pairs)
